# initial kernel scaffold (unmeasured)
import jax
import jax.numpy as jnp
from jax import lax
from jax.experimental import pallas as pl
from jax.experimental.pallas import tpu as pltpu

N_DEV = 4
E_PER = 2
N_EXP = N_DEV * E_PER


def kernel(
    x,
    router_W,
    route_idx,
    expert_W,
    shared_W,
):
    n_tok, d = x.shape
    e_per, _, h = expert_W.shape

    def body(x_ref, rw_ref, idx_ref, ew_ref, sw_ref, out_ref,
             comm_ref, send_sems, recv_sems):
        my_pos = lax.axis_index("i")

        comm_ref[0] = ew_ref[...].astype(jnp.bfloat16)

        barrier_sem = pltpu.get_barrier_semaphore()
        for k in range(1, N_DEV):
            pl.semaphore_signal(
                barrier_sem, inc=1,
                device_id=(lax.rem(my_pos + k, N_DEV),),
                device_id_type=pl.DeviceIdType.MESH,
            )
        pl.semaphore_wait(barrier_sem, N_DEV - 1)

        rdmas = []
        for k in range(1, N_DEV):
            rdma = pltpu.make_async_remote_copy(
                src_ref=comm_ref.at[0],
                dst_ref=comm_ref.at[k],
                send_sem=send_sems.at[k - 1],
                recv_sem=recv_sems.at[k - 1],
                device_id=(lax.rem(my_pos + k, N_DEV),),
                device_id_type=pl.DeviceIdType.MESH,
            )
            rdma.start()
            rdmas.append(rdma)

        xf = x_ref[...]
        scores = jnp.dot(xf, rw_ref[...], preferred_element_type=jnp.float32)
        scores = scores - jnp.max(scores, axis=-1, keepdims=True)
        p = jnp.exp(scores)
        probs = p / jnp.sum(p, axis=-1, keepdims=True)
        eids = lax.broadcasted_iota(jnp.int32, (n_tok, N_EXP), 1)
        coeff = jnp.where(eids == idx_ref[...], probs, 0.0)

        xb = xf.astype(jnp.bfloat16)
        acc = jnp.dot(xb, sw_ref[...].astype(jnp.bfloat16),
                      preferred_element_type=jnp.float32)

        def add_pair(acc, slot):
            src = lax.rem(my_pos - slot + N_DEV, N_DEV)
            cb = lax.dynamic_slice(coeff, (0, src * E_PER), (n_tok, E_PER))
            for j in range(E_PER):
                y = jnp.dot(xb, comm_ref[slot, j],
                            preferred_element_type=jnp.float32)
                acc = acc + cb[:, j:j + 1] * y
            return acc

        acc = add_pair(acc, 0)

        for k in range(1, N_DEV):
            rdmas[k - 1].wait_recv()
            acc = add_pair(acc, k)

        out_ref[...] = acc

        for rdma in rdmas:
            rdma.wait_send()

    return pl.pallas_call(
        body,
        out_shape=jax.ShapeDtypeStruct((n_tok, h), jnp.float32),
        in_specs=[pl.BlockSpec(memory_space=pltpu.VMEM)] * 5,
        out_specs=pl.BlockSpec(memory_space=pltpu.VMEM),
        scratch_shapes=[
            pltpu.VMEM((N_DEV, e_per, d, h), jnp.bfloat16),
            pltpu.SemaphoreType.DMA((N_DEV - 1,)),
            pltpu.SemaphoreType.DMA((N_DEV - 1,)),
        ],
        compiler_params=pltpu.CompilerParams(collective_id=0),
    )(x, router_W, route_idx, expert_W, shared_W)


# baseline (device time: 22141 ns/iter reference)
import jax
import jax.numpy as jnp
from jax import lax
from jax.experimental import pallas as pl
from jax.experimental.pallas import tpu as pltpu

N_DEV = 4
E_PER = 2
N_EXP = N_DEV * E_PER


def kernel(
    x,
    router_W,
    route_idx,
    expert_W,
    shared_W,
):
    n_tok, d = x.shape
    e_per, _, h = expert_W.shape

    def body(x_ref, rw_ref, idx_ref, ew_ref, sw_ref, out_ref,
             comm_ref, send_sems, recv_sems):
        my_pos = lax.axis_index("i")

        comm_ref[0] = ew_ref[...].astype(jnp.bfloat16)

        barrier_sem = pltpu.get_barrier_semaphore()
        for k in range(1, N_DEV):
            pl.semaphore_signal(
                barrier_sem, inc=1,
                device_id=(lax.rem(my_pos + k, N_DEV),),
                device_id_type=pl.DeviceIdType.MESH,
            )
        pl.semaphore_wait(barrier_sem, N_DEV - 1)

        rdmas = []
        for k in range(1, N_DEV):
            rdma = pltpu.make_async_remote_copy(
                src_ref=comm_ref.at[0],
                dst_ref=comm_ref.at[k],
                send_sem=send_sems.at[k - 1],
                recv_sem=recv_sems.at[k - 1],
                device_id=(lax.rem(my_pos + k, N_DEV),),
                device_id_type=pl.DeviceIdType.MESH,
            )
            rdma.start()
            rdmas.append(rdma)

        xf = x_ref[...]
        scores = jnp.dot(xf, rw_ref[...], preferred_element_type=jnp.float32)
        scores = scores - jnp.max(scores, axis=-1, keepdims=True)
        p = jnp.exp(scores)
        probs = p / jnp.sum(p, axis=-1, keepdims=True)
        eids = lax.broadcasted_iota(jnp.int32, (n_tok, N_EXP), 1)
        coeff = jnp.where(eids == idx_ref[...], probs, 0.0)

        xb = xf.astype(jnp.bfloat16)
        acc = jnp.dot(xb, sw_ref[...].astype(jnp.bfloat16),
                      preferred_element_type=jnp.float32)

        def add_pair(acc, slot):
            src = lax.rem(my_pos - slot + N_DEV, N_DEV)
            for j in range(E_PER):
                e = src * E_PER + j
                cj = jnp.sum(jnp.where(eids == e, coeff, 0.0),
                             axis=1, keepdims=True)
                y = jnp.dot(xb, comm_ref[slot, j],
                            preferred_element_type=jnp.float32)
                acc = acc + cj * y
            return acc

        acc = add_pair(acc, 0)

        for k in range(1, N_DEV):
            rdmas[k - 1].wait_recv()
            acc = add_pair(acc, k)

        out_ref[...] = acc

        for rdma in rdmas:
            rdma.wait_send()

    return pl.pallas_call(
        body,
        out_shape=jax.ShapeDtypeStruct((n_tok, h), jnp.float32),
        in_specs=[pl.BlockSpec(memory_space=pltpu.VMEM)] * 5,
        out_specs=pl.BlockSpec(memory_space=pltpu.VMEM),
        scratch_shapes=[
            pltpu.VMEM((N_DEV, e_per, d, h), jnp.bfloat16),
            pltpu.SemaphoreType.DMA((N_DEV - 1,)),
            pltpu.SemaphoreType.DMA((N_DEV - 1,)),
        ],
        compiler_params=pltpu.CompilerParams(collective_id=0),
    )(x, router_W, route_idx, expert_W, shared_W)


# device time: 19453 ns/iter; 1.1382x vs baseline; 1.1382x over previous
import jax
import jax.numpy as jnp
from jax import lax
from jax.experimental import pallas as pl
from jax.experimental.pallas import tpu as pltpu

N_DEV = 4
E_PER = 2
N_EXP = N_DEV * E_PER
CAP = 192


def kernel(
    x,
    router_W,
    route_idx,
    expert_W,
    shared_W,
):
    n_tok, d = x.shape
    e_per, _, h = expert_W.shape

    def body(x_ref, rw_ref, idx_ref, ew_ref, sw_ref, out_ref,
             xs_ref, xr_ref, bs_ref, br_ref, ys_ref, yr_ref,
             s1x, r1x, s1b, r1b, s2, r2):
        my_pos = lax.axis_index("i")

        barrier_sem = pltpu.get_barrier_semaphore()
        for k in range(1, N_DEV):
            pl.semaphore_signal(
                barrier_sem, inc=1,
                device_id=(lax.rem(my_pos + k, N_DEV),),
                device_id_type=pl.DeviceIdType.MESH,
            )
        pl.semaphore_wait(barrier_sem, N_DEV - 1)

        idx = idx_ref[...]
        g = lax.div(idx, E_PER)
        b = (idx - g * E_PER).astype(jnp.float32)

        ts = lax.rem(my_pos + lax.broadcasted_iota(jnp.int32, (1, N_DEV), 1),
                     N_DEV)
        M = (g == ts).astype(jnp.float32)
        r_io = lax.broadcasted_iota(jnp.int32, (n_tok, n_tok), 0)
        c_io = lax.broadcasted_iota(jnp.int32, (n_tok, n_tok), 1)
        tri = (r_io >= c_io).astype(jnp.float32)
        rank = jnp.dot(tri, M, preferred_element_type=jnp.float32) - 1.0

        xf = x_ref[...]
        xb = xf.astype(jnp.bfloat16)
        cap_io = lax.broadcasted_iota(jnp.int32, (n_tok, CAP), 1)

        rdmas = []
        PTs = [None] * N_DEV
        for k in range(1, N_DEV):
            mk = M[:, k:k + 1]
            rk = rank[:, k:k + 1].astype(jnp.int32)
            PT = ((cap_io == rk).astype(jnp.float32) * mk
                  ).astype(jnp.bfloat16)
            PTs[k] = PT
            xs_ref[k - 1] = lax.dot_general(
                PT, xb, (((0,), (0,)), ((), ())),
                preferred_element_type=jnp.float32,
                ).astype(jnp.bfloat16)
            bs_ref[k - 1] = lax.dot_general(
                b, PT.astype(jnp.float32), (((0,), (0,)), ((), ())),
                preferred_element_type=jnp.float32)
            dev = (lax.rem(my_pos + k, N_DEV),)
            rx = pltpu.make_async_remote_copy(
                src_ref=xs_ref.at[k - 1], dst_ref=xr_ref.at[k - 1],
                send_sem=s1x.at[k - 1], recv_sem=r1x.at[k - 1],
                device_id=dev, device_id_type=pl.DeviceIdType.MESH)
            rb = pltpu.make_async_remote_copy(
                src_ref=bs_ref.at[k - 1], dst_ref=br_ref.at[k - 1],
                send_sem=s1b.at[k - 1], recv_sem=r1b.at[k - 1],
                device_id=dev, device_id_type=pl.DeviceIdType.MESH)
            rx.start()
            rb.start()
            rdmas.append((rx, rb))

        scores = jnp.dot(xf, rw_ref[...], preferred_element_type=jnp.float32)
        scores = scores - jnp.max(scores, axis=-1, keepdims=True)
        p = jnp.exp(scores)
        probs = p / jnp.sum(p, axis=-1, keepdims=True)
        eids = lax.broadcasted_iota(jnp.int32, (n_tok, N_EXP), 1)
        c = jnp.sum(jnp.where(eids == idx, probs, 0.0),
                    axis=1, keepdims=True)

        ewb = ew_ref[...].astype(jnp.bfloat16)
        acc = jnp.dot(xb, sw_ref[...].astype(jnp.bfloat16),
                      preferred_element_type=jnp.float32)
        y0 = jnp.dot(xb, ewb[0], preferred_element_type=jnp.float32)
        y1 = jnp.dot(xb, ewb[1], preferred_element_type=jnp.float32)
        ysel = jnp.where(b > 0.5, y1, y0)
        acc = acc + (c * M[:, 0:1]) * ysel

        i_r = lax.broadcasted_iota(jnp.int32, (CAP, CAP), 0)
        i_c = lax.broadcasted_iota(jnp.int32, (CAP, CAP), 1)
        eye = (i_r == i_c).astype(jnp.float32)

        order = (1, 3, 2)
        ret = []
        for k in order:
            rx, rb = rdmas[k - 1]
            rx.wait_recv()
            rb.wait_recv()
            xr = xr_ref[k - 1]
            bits = lax.dot_general(
                eye, br_ref[k - 1], (((1,), (1,)), ((), ())),
                preferred_element_type=jnp.float32)
            z0 = jnp.dot(xr, ewb[0], preferred_element_type=jnp.float32)
            z1 = jnp.dot(xr, ewb[1], preferred_element_type=jnp.float32)
            ys_ref[k - 1] = jnp.where(bits > 0.5, z1, z0).astype(jnp.bfloat16)
            ry = pltpu.make_async_remote_copy(
                src_ref=ys_ref.at[k - 1], dst_ref=yr_ref.at[k - 1],
                send_sem=s2.at[k - 1], recv_sem=r2.at[k - 1],
                device_id=(lax.rem(my_pos - k + N_DEV, N_DEV),),
                device_id_type=pl.DeviceIdType.MESH)
            ry.start()
            ret.append(ry)

        for k in order:
            ret[order.index(k)].wait_recv()
            ysc = jnp.dot(PTs[k], yr_ref[k - 1],
                          preferred_element_type=jnp.float32)
            acc = acc + c * ysc

        out_ref[...] = acc

        for rx, rb in rdmas:
            rx.wait_send()
            rb.wait_send()
        for ry in ret:
            ry.wait_send()

    return pl.pallas_call(
        body,
        out_shape=jax.ShapeDtypeStruct((n_tok, h), jnp.float32),
        in_specs=[pl.BlockSpec(memory_space=pltpu.VMEM)] * 5,
        out_specs=pl.BlockSpec(memory_space=pltpu.VMEM),
        scratch_shapes=[
            pltpu.VMEM((N_DEV - 1, CAP, d), jnp.bfloat16),
            pltpu.VMEM((N_DEV - 1, CAP, d), jnp.bfloat16),
            pltpu.VMEM((N_DEV - 1, 1, CAP), jnp.float32),
            pltpu.VMEM((N_DEV - 1, 1, CAP), jnp.float32),
            pltpu.VMEM((N_DEV - 1, CAP, h), jnp.bfloat16),
            pltpu.VMEM((N_DEV - 1, CAP, h), jnp.bfloat16),
            pltpu.SemaphoreType.DMA((N_DEV - 1,)),
            pltpu.SemaphoreType.DMA((N_DEV - 1,)),
            pltpu.SemaphoreType.DMA((N_DEV - 1,)),
            pltpu.SemaphoreType.DMA((N_DEV - 1,)),
            pltpu.SemaphoreType.DMA((N_DEV - 1,)),
            pltpu.SemaphoreType.DMA((N_DEV - 1,)),
        ],
        compiler_params=pltpu.CompilerParams(collective_id=0),
    )(x, router_W, route_idx, expert_W, shared_W)
